# baseline (speedup 1.0000x reference)
import functools

import jax
import jax.numpy as jnp
from jax import lax
from jax.experimental import pallas as pl
from jax.experimental.pallas import tpu as pltpu
from jax.experimental.pallas import tpu_sc as plsc

N = 10000
E = 320000
HID = 128
L = 6
NB = 1000
GRID = N // NB

NC = 2
NS = 16
NW = NC * NS
EPW = E // NW
EK = 80
NCHUNK = EPW // EK
RPS = N // NS


def _sigmoid(x):
    return 1.0 / (1.0 + jnp.exp(-x))


def _hi_lo(m):
    hi = (m + 6144.0) - 6144.0
    return hi, m - hi


def _proj_body(x_ref, wpT_ref, b_ref, w0_ref, h_ref, mh_ref, ml_ref):
    h = jnp.dot(x_ref[...], wpT_ref[...], preferred_element_type=jnp.float32)
    h = h + b_ref[...]
    h_ref[...] = h
    m = jnp.dot(h, w0_ref[...], preferred_element_type=jnp.float32)
    mh_ref[...], ml_ref[...] = _hi_lo(m)


def _proj(x, wpT, b2d, w0):
    return pl.pallas_call(
        _proj_body,
        grid=(GRID,),
        in_specs=[
            pl.BlockSpec((NB, HID), lambda i: (i, 0)),
            pl.BlockSpec((HID, HID), lambda i: (0, 0)),
            pl.BlockSpec((1, HID), lambda i: (0, 0)),
            pl.BlockSpec((HID, HID), lambda i: (0, 0)),
        ],
        out_specs=[
            pl.BlockSpec((NB, HID), lambda i: (i, 0)),
            pl.BlockSpec((NB, HID), lambda i: (i, 0)),
            pl.BlockSpec((NB, HID), lambda i: (i, 0)),
        ],
        out_shape=[
            jax.ShapeDtypeStruct((N, HID), jnp.float32),
            jax.ShapeDtypeStruct((N, HID), jnp.float32),
            jax.ShapeDtypeStruct((N, HID), jnp.float32),
        ],
    )(x, wpT, b2d, w0)


def _gru_body(h_ref, p0h_ref, p1h_ref, p0l_ref, p1l_ref, wihT_ref, whhT_ref,
              bih_ref, bhh_ref, wn_ref, hn_ref, mh_ref, ml_ref):
    h = h_ref[...]
    agg = (p0h_ref[...] + p1h_ref[...]) + (p0l_ref[...] + p1l_ref[...])
    gi = jnp.dot(agg, wihT_ref[...], preferred_element_type=jnp.float32)
    gi = gi + bih_ref[...]
    gh = jnp.dot(h, whhT_ref[...], preferred_element_type=jnp.float32)
    gh = gh + bhh_ref[...]
    r = _sigmoid(gi[:, :HID] + gh[:, :HID])
    z = _sigmoid(gi[:, HID:2 * HID] + gh[:, HID:2 * HID])
    n = jnp.tanh(gi[:, 2 * HID:] + r * gh[:, 2 * HID:])
    hn = (1.0 - z) * n + z * h
    hn_ref[...] = hn
    m = jnp.dot(hn, wn_ref[...], preferred_element_type=jnp.float32)
    mh_ref[...], ml_ref[...] = _hi_lo(m)


def _gru_step(h, p0h, p1h, p0l, p1l, wihT, whhT, bih2d, bhh2d, wn):
    return pl.pallas_call(
        _gru_body,
        grid=(GRID,),
        in_specs=[
            pl.BlockSpec((NB, HID), lambda i: (i, 0)),
            pl.BlockSpec((NB, HID), lambda i: (i, 0)),
            pl.BlockSpec((NB, HID), lambda i: (i, 0)),
            pl.BlockSpec((NB, HID), lambda i: (i, 0)),
            pl.BlockSpec((NB, HID), lambda i: (i, 0)),
            pl.BlockSpec((HID, 3 * HID), lambda i: (0, 0)),
            pl.BlockSpec((HID, 3 * HID), lambda i: (0, 0)),
            pl.BlockSpec((1, 3 * HID), lambda i: (0, 0)),
            pl.BlockSpec((1, 3 * HID), lambda i: (0, 0)),
            pl.BlockSpec((HID, HID), lambda i: (0, 0)),
        ],
        out_specs=[
            pl.BlockSpec((NB, HID), lambda i: (i, 0)),
            pl.BlockSpec((NB, HID), lambda i: (i, 0)),
            pl.BlockSpec((NB, HID), lambda i: (i, 0)),
        ],
        out_shape=[
            jax.ShapeDtypeStruct((N, HID), jnp.float32),
            jax.ShapeDtypeStruct((N, HID), jnp.float32),
            jax.ShapeDtypeStruct((N, HID), jnp.float32),
        ],
    )(h, p0h, p1h, p0l, p1l, wihT, whhT, bih2d, bhh2d, wn)


RCH = 80
NRCH = N // RCH
ZITER = (NRCH + NS - 1) // NS


def _sc_scatter_body(m_hbm, src_hbm, dst_hbm, zero_hbm, tok_hbm, out_hbm,
                     src_v, dst_v, dstbuf, buf, acc, sem):
    del tok_hbm
    c = lax.axis_index("c")
    s = lax.axis_index("s")
    wid = s * NC + c
    pltpu.sync_copy(src_hbm.at[wid], src_v)
    pltpu.sync_copy(dst_hbm.at[wid], dst_v)

    def zbody(t, carry):
        j = s + t * NS

        @pl.when(j < NRCH)
        def _():
            pltpu.sync_copy(zero_hbm.at[pl.ds(j * RCH, RCH)],
                            acc.at[pl.ds(j * RCH, RCH)])

        return carry

    lax.fori_loop(0, ZITER, zbody, 0)
    plsc.subcore_barrier()

    def body(j, carry):
        pltpu.async_copy(m_hbm.at[src_v.at[j]], buf, sem).wait()
        for k in range(EK // 16):
            dstbuf[pl.ds(k * 16, 16)] = dst_v[j, pl.ds(k * 16, 16)]
        pltpu.sync_copy(buf, acc.at[dstbuf], add=True)
        return carry

    lax.fori_loop(0, NCHUNK, body, 0)
    plsc.subcore_barrier()

    def fbody(t, carry):
        j = s + t * NS

        @pl.when(j < NRCH)
        def _():
            pltpu.sync_copy(acc.at[pl.ds(j * RCH, RCH)],
                            out_hbm.at[c, pl.ds(j * RCH, RCH)])

        return carry

    lax.fori_loop(0, ZITER, fbody, 0)


def _sc_scatter(m, src_r3, dst_r3, zeros_nh, tok):
    fn = pl.kernel(
        _sc_scatter_body,
        out_type=jax.ShapeDtypeStruct((NC, N, HID), jnp.float32),
        mesh=plsc.VectorSubcoreMesh(core_axis_name="c", subcore_axis_name="s",
                                    num_cores=NC, num_subcores=NS),
        scratch_types=[
            pltpu.VMEM((NCHUNK, EK), jnp.int32),
            pltpu.VMEM((NCHUNK, EK), jnp.int32),
            pltpu.VMEM((EK,), jnp.int32),
            pltpu.VMEM((EK, HID), jnp.float32),
            pltpu.VMEM_SHARED((N, HID), jnp.float32),
            pltpu.SemaphoreType.DMA,
        ],
    )
    return fn(m, src_r3, dst_r3, zeros_nh, tok)


def _conv1d(x, w, b, pad):
    out = lax.conv_general_dilated(
        x, w, window_strides=(1,), padding=[(pad, pad)],
        dimension_numbers=('NCH', 'OIH', 'NCH'))
    return out + b[None, :, None]


def _maxpool1d(x, k, s):
    return lax.reduce_window(x, -jnp.inf, lax.max, (1, 1, k), (1, 1, s),
                             'VALID')


def _readout(h, x, conv1_w, conv1_b, conv2_w, conv2_b, fc1_w, fc1_b, fc2_w,
             fc2_b):
    concat = jnp.concatenate([h, x], axis=1)[:, None, :]
    Z = _maxpool1d(jax.nn.relu(_conv1d(concat, conv1_w, conv1_b, 1)), 3, 2)
    Z = _maxpool1d(_conv1d(Z, conv2_w, conv2_b, 1), 2, 2)
    hh = h[:, None, :]
    Y = _maxpool1d(jax.nn.relu(_conv1d(hh, conv1_w, conv1_b, 1)), 3, 2)
    Y = _maxpool1d(_conv1d(Y, conv2_w, conv2_b, 1), 2, 2)
    Zf = Z.reshape(N, -1)
    Yf = Y.reshape(N, -1)
    res = (Zf @ fc1_w.T + fc1_b) * (Yf @ fc2_w.T + fc2_b)
    p = jax.nn.sigmoid(res.reshape(-1))
    eps = 1e-6
    p = jnp.clip(p, eps, 1.0 - eps)
    z1 = jnp.log(p / (1.0 - p))
    return jnp.stack([jnp.zeros_like(z1), z1], axis=1)


def kernel(x, edge_index, w_proj, b_proj, ggnn_w, gru_wih, gru_whh, gru_bih,
           gru_bhh, conv1_w, conv1_b, conv2_w, conv2_b, fc1_w, fc1_b, fc2_w,
           fc2_b):
    order = jnp.argsort(edge_index[1])
    src_s = edge_index[0][order].reshape(EK, E // EK).T
    dst_s = edge_index[1][order].reshape(EK, E // EK).T
    src_r = src_s.reshape(NW, NCHUNK, EK)
    dst_r = dst_s.reshape(NW, NCHUNK, EK)
    zeros_nh = jnp.zeros((N, HID), jnp.float32)
    wpT = w_proj.T
    b2d = b_proj.reshape(1, HID)
    wihT = gru_wih.T
    whhT = gru_whh.T
    bih2d = gru_bih.reshape(1, 3 * HID)
    bhh2d = gru_bhh.reshape(1, 3 * HID)

    h, mh, ml = _proj(x, wpT, b2d, ggnn_w[0])
    for l in range(L):
        m = mh + ml
        agg = jnp.zeros((N, HID), jnp.float32).at[edge_index[1]].add(
            m[edge_index[0]])
        wn = ggnn_w[l + 1] if l + 1 < L else ggnn_w[0]
        h, mh, ml = _gru_step(h, agg, zeros_nh, zeros_nh, zeros_nh, wihT,
                              whhT, bih2d, bhh2d, wn)

    return _readout(h, x, conv1_w, conv1_b, conv2_w, conv2_b, fc1_w, fc1_b,
                    fc2_w, fc2_b)


# speedup vs baseline: 1.0683x; 1.0683x over previous
import functools

import jax
import jax.numpy as jnp
from jax import lax
from jax.experimental import pallas as pl
from jax.experimental.pallas import tpu as pltpu
from jax.experimental.pallas import tpu_sc as plsc

N = 10000
E = 320000
HID = 128
L = 6
NB = 1000
GRID = N // NB

NC = 2
NS = 16
NW = NC * NS
EPW = E // NW
EK = 80
NCHUNK = EPW // EK
RPS = N // NS


def _sigmoid(x):
    return 1.0 / (1.0 + jnp.exp(-x))


def _hi_lo(m):
    hi = (m + 6144.0) - 6144.0
    return hi, m - hi


def _proj_body(x_ref, wpT_ref, b_ref, w0_ref, h_ref, m_ref):
    h = jnp.dot(x_ref[...], wpT_ref[...], preferred_element_type=jnp.float32)
    h = h + b_ref[...]
    h_ref[...] = h
    m_ref[...] = jnp.dot(h, w0_ref[...], preferred_element_type=jnp.float32)


def _proj(x, wpT, b2d, w0):
    return pl.pallas_call(
        _proj_body,
        grid=(GRID,),
        in_specs=[
            pl.BlockSpec((NB, HID), lambda i: (i, 0)),
            pl.BlockSpec((HID, HID), lambda i: (0, 0)),
            pl.BlockSpec((1, HID), lambda i: (0, 0)),
            pl.BlockSpec((HID, HID), lambda i: (0, 0)),
        ],
        out_specs=[
            pl.BlockSpec((NB, HID), lambda i: (i, 0)),
            pl.BlockSpec((NB, HID), lambda i: (i, 0)),
        ],
        out_shape=[
            jax.ShapeDtypeStruct((N, HID), jnp.float32),
            jax.ShapeDtypeStruct((N, HID), jnp.float32),
        ],
    )(x, wpT, b2d, w0)


def _gru_body(h_ref, agg_ref, wihT_ref, whhT_ref,
              bih_ref, bhh_ref, wn_ref, hn_ref, mn_ref):
    h = h_ref[...]
    agg = agg_ref[...]
    gi = jnp.dot(agg, wihT_ref[...], preferred_element_type=jnp.float32)
    gi = gi + bih_ref[...]
    gh = jnp.dot(h, whhT_ref[...], preferred_element_type=jnp.float32)
    gh = gh + bhh_ref[...]
    r = _sigmoid(gi[:, :HID] + gh[:, :HID])
    z = _sigmoid(gi[:, HID:2 * HID] + gh[:, HID:2 * HID])
    n = jnp.tanh(gi[:, 2 * HID:] + r * gh[:, 2 * HID:])
    hn = (1.0 - z) * n + z * h
    hn_ref[...] = hn
    mn_ref[...] = jnp.dot(hn, wn_ref[...], preferred_element_type=jnp.float32)


def _gru_step(h, agg, wihT, whhT, bih2d, bhh2d, wn):
    return pl.pallas_call(
        _gru_body,
        grid=(GRID,),
        in_specs=[
            pl.BlockSpec((NB, HID), lambda i: (i, 0)),
            pl.BlockSpec((NB, HID), lambda i: (i, 0)),
            pl.BlockSpec((HID, 3 * HID), lambda i: (0, 0)),
            pl.BlockSpec((HID, 3 * HID), lambda i: (0, 0)),
            pl.BlockSpec((1, 3 * HID), lambda i: (0, 0)),
            pl.BlockSpec((1, 3 * HID), lambda i: (0, 0)),
            pl.BlockSpec((HID, HID), lambda i: (0, 0)),
        ],
        out_specs=[
            pl.BlockSpec((NB, HID), lambda i: (i, 0)),
            pl.BlockSpec((NB, HID), lambda i: (i, 0)),
        ],
        out_shape=[
            jax.ShapeDtypeStruct((N, HID), jnp.float32),
            jax.ShapeDtypeStruct((N, HID), jnp.float32),
        ],
    )(h, agg, wihT, whhT, bih2d, bhh2d, wn)


RCH = 80
NRCH = N // RCH
ZITER = (NRCH + NS - 1) // NS


def _sc_scatter_body(m_hbm, src_hbm, dst_hbm, zero_hbm, tok_hbm, out_hbm,
                     src_v, dst_v, dstbuf, buf, acc, sem):
    del tok_hbm
    c = lax.axis_index("c")
    s = lax.axis_index("s")
    wid = s * NC + c
    pltpu.sync_copy(src_hbm.at[wid], src_v)
    pltpu.sync_copy(dst_hbm.at[wid], dst_v)

    def zbody(t, carry):
        j = s + t * NS

        @pl.when(j < NRCH)
        def _():
            pltpu.sync_copy(zero_hbm.at[pl.ds(j * RCH, RCH)],
                            acc.at[pl.ds(j * RCH, RCH)])

        return carry

    lax.fori_loop(0, ZITER, zbody, 0)
    plsc.subcore_barrier()

    def body(j, carry):
        pltpu.async_copy(m_hbm.at[src_v.at[j]], buf, sem).wait()
        for k in range(EK // 16):
            dstbuf[pl.ds(k * 16, 16)] = dst_v[j, pl.ds(k * 16, 16)]
        pltpu.sync_copy(buf, acc.at[dstbuf], add=True)
        return carry

    lax.fori_loop(0, NCHUNK, body, 0)
    plsc.subcore_barrier()

    def fbody(t, carry):
        j = s + t * NS

        @pl.when(j < NRCH)
        def _():
            pltpu.sync_copy(acc.at[pl.ds(j * RCH, RCH)],
                            out_hbm.at[c, pl.ds(j * RCH, RCH)])

        return carry

    lax.fori_loop(0, ZITER, fbody, 0)


def _sc_scatter(m, src_r3, dst_r3, zeros_nh, tok):
    fn = pl.kernel(
        _sc_scatter_body,
        out_type=jax.ShapeDtypeStruct((NC, N, HID), jnp.float32),
        mesh=plsc.VectorSubcoreMesh(core_axis_name="c", subcore_axis_name="s",
                                    num_cores=NC, num_subcores=NS),
        scratch_types=[
            pltpu.VMEM((NCHUNK, EK), jnp.int32),
            pltpu.VMEM((NCHUNK, EK), jnp.int32),
            pltpu.VMEM((EK,), jnp.int32),
            pltpu.VMEM((EK, HID), jnp.float32),
            pltpu.VMEM_SHARED((N, HID), jnp.float32),
            pltpu.SemaphoreType.DMA,
        ],
    )
    return fn(m, src_r3, dst_r3, zeros_nh, tok)


def _conv1d(x, w, b, pad):
    out = lax.conv_general_dilated(
        x, w, window_strides=(1,), padding=[(pad, pad)],
        dimension_numbers=('NCH', 'OIH', 'NCH'))
    return out + b[None, :, None]


def _maxpool1d(x, k, s):
    return lax.reduce_window(x, -jnp.inf, lax.max, (1, 1, k), (1, 1, s),
                             'VALID')


def _readout(h, x, conv1_w, conv1_b, conv2_w, conv2_b, fc1_w, fc1_b, fc2_w,
             fc2_b):
    concat = jnp.concatenate([h, x], axis=1)[:, None, :]
    Z = _maxpool1d(jax.nn.relu(_conv1d(concat, conv1_w, conv1_b, 1)), 3, 2)
    Z = _maxpool1d(_conv1d(Z, conv2_w, conv2_b, 1), 2, 2)
    hh = h[:, None, :]
    Y = _maxpool1d(jax.nn.relu(_conv1d(hh, conv1_w, conv1_b, 1)), 3, 2)
    Y = _maxpool1d(_conv1d(Y, conv2_w, conv2_b, 1), 2, 2)
    Zf = Z.reshape(N, -1)
    Yf = Y.reshape(N, -1)
    res = (Zf @ fc1_w.T + fc1_b) * (Yf @ fc2_w.T + fc2_b)
    p = jax.nn.sigmoid(res.reshape(-1))
    eps = 1e-6
    p = jnp.clip(p, eps, 1.0 - eps)
    z1 = jnp.log(p / (1.0 - p))
    return jnp.stack([jnp.zeros_like(z1), z1], axis=1)


def kernel(x, edge_index, w_proj, b_proj, ggnn_w, gru_wih, gru_whh, gru_bih,
           gru_bhh, conv1_w, conv1_b, conv2_w, conv2_b, fc1_w, fc1_b, fc2_w,
           fc2_b):
    order = jnp.argsort(edge_index[1])
    src_s = edge_index[0][order].reshape(EK, E // EK).T
    dst_s = edge_index[1][order].reshape(EK, E // EK).T
    src_r = src_s.reshape(NW, NCHUNK, EK)
    dst_r = dst_s.reshape(NW, NCHUNK, EK)
    zeros_nh = jnp.zeros((N, HID), jnp.float32)
    wpT = w_proj.T
    b2d = b_proj.reshape(1, HID)
    wihT = gru_wih.T
    whhT = gru_whh.T
    bih2d = gru_bih.reshape(1, 3 * HID)
    bhh2d = gru_bhh.reshape(1, 3 * HID)

    h, m = _proj(x, wpT, b2d, ggnn_w[0])
    for l in range(L):
        agg = jnp.zeros((N, HID), jnp.float32).at[edge_index[1]].add(
            m[edge_index[0]])
        wn = ggnn_w[l + 1] if l + 1 < L else ggnn_w[0]
        h, m = _gru_step(h, agg, wihT, whhT, bih2d, bhh2d, wn)

    return _readout(h, x, conv1_w, conv1_b, conv2_w, conv2_b, fc1_w, fc1_b,
                    fc2_w, fc2_b)
